# Initial kernel scaffold; baseline (speedup 1.0000x reference)
import jax
import jax.numpy as jnp
from jax.experimental import pallas as pl


def kernel(x, edge_index, Ws, bs, alphas):
    raise NotImplementedError("write your pallas kernel here")


# speedup vs baseline: 1.6571x; 1.6571x over previous
import functools

import jax
import jax.numpy as jnp
from jax.experimental import pallas as pl
from jax.experimental.pallas import tpu as pltpu

N = 10000
D = 128
BI = 200
NI = N // BI


def _gcn_pass_kernel(n_out, adj_ref, h_ref, w_ref, b_ref, a_ref, *refs):
    o_refs = refs[:n_out]
    seq_ref = refs[n_out]
    i = pl.program_id(0)

    @pl.when(i == 0)
    def _():
        seq_ref[...] = jnp.dot(h_ref[...], w_ref[...],
                               preferred_element_type=jnp.float32)

    z = jnp.dot(adj_ref[...], seq_ref[...],
                preferred_element_type=jnp.float32)
    z = z + b_ref[...]
    z = jnp.where(z >= 0, z, a_ref[...] * z)
    z = jnp.maximum(z, 0.0)
    for j in range(n_out):
        o_refs[j][...] = z[:, j * D:(j + 1) * D]


def _gcn_pass(adj, h, W, b, a, n_out):
    F = W.shape[1]
    return pl.pallas_call(
        functools.partial(_gcn_pass_kernel, n_out),
        grid=(NI,),
        in_specs=[
            pl.BlockSpec((BI, N), lambda i: (i, 0)),
            pl.BlockSpec((N, D), lambda i: (0, 0)),
            pl.BlockSpec((D, F), lambda i: (0, 0)),
            pl.BlockSpec((1, F), lambda i: (0, 0)),
            pl.BlockSpec((1, F), lambda i: (0, 0)),
        ],
        out_specs=[pl.BlockSpec((BI, D), lambda i: (i, 0))] * n_out,
        out_shape=[jax.ShapeDtypeStruct((N, D), jnp.float32)] * n_out,
        scratch_shapes=[pltpu.VMEM((N, F), jnp.float32)],
        compiler_params=pltpu.CompilerParams(
            dimension_semantics=("arbitrary",)),
    )(adj, h, W, b, a)


def kernel(x, edge_index, Ws, bs, alphas):
    adj = edge_index
    h = x
    for i in range(3):
        (h,) = _gcn_pass(adj, h, Ws[i], bs[i][None, :],
                         jnp.full((1, D), alphas[i], dtype=jnp.float32), 1)
    Wcat = jnp.concatenate([Ws[3], Ws[4], Ws[5], Ws[6]], axis=1)
    bcat = jnp.concatenate([bs[3], bs[4], bs[5], bs[6]])[None, :]
    acat = jnp.repeat(alphas[3:7], D)[None, :]
    outs = _gcn_pass(adj, h, Wcat, bcat, acat, 4)
    return tuple(outs)


# speedup vs baseline: 1.7331x; 1.0459x over previous
import functools

import jax
import jax.numpy as jnp
from jax.experimental import pallas as pl
from jax.experimental.pallas import tpu as pltpu

N = 10000
D = 128
BI = 200
NI = N // BI


def _first_pass_kernel(adj_ref, h_ref, w_ref, b_ref, a_ref,
                       o_ref, adj_bf_ref, seq_ref):
    i = pl.program_id(0)

    @pl.when(i == 0)
    def _():
        seq_ref[...] = jnp.dot(h_ref[...], w_ref[...],
                               preferred_element_type=jnp.float32)

    adj_blk = adj_ref[...]
    adj_bf_ref[...] = adj_blk.astype(jnp.bfloat16)
    z = jnp.dot(adj_blk, seq_ref[...], preferred_element_type=jnp.float32)
    z = z + b_ref[...]
    z = jnp.where(z >= 0, z, a_ref[...] * z)
    o_ref[...] = jnp.maximum(z, 0.0)


def _bf16_pass_kernel(n_out, adj_ref, h_ref, w_ref, b_ref, a_ref, *refs):
    o_refs = refs[:n_out]
    seq_ref = refs[n_out]
    i = pl.program_id(0)

    @pl.when(i == 0)
    def _():
        seq_ref[...] = jnp.dot(
            h_ref[...], w_ref[...],
            preferred_element_type=jnp.float32).astype(jnp.bfloat16)

    z = jnp.dot(adj_ref[...], seq_ref[...],
                preferred_element_type=jnp.float32)
    z = z + b_ref[...]
    z = jnp.where(z >= 0, z, a_ref[...] * z)
    z = jnp.maximum(z, 0.0)
    for j in range(n_out):
        o_refs[j][...] = z[:, j * D:(j + 1) * D]


def _first_pass(adj, h, W, b, a):
    return pl.pallas_call(
        _first_pass_kernel,
        grid=(NI,),
        in_specs=[
            pl.BlockSpec((BI, N), lambda i: (i, 0)),
            pl.BlockSpec((N, D), lambda i: (0, 0)),
            pl.BlockSpec((D, D), lambda i: (0, 0)),
            pl.BlockSpec((1, D), lambda i: (0, 0)),
            pl.BlockSpec((1, D), lambda i: (0, 0)),
        ],
        out_specs=[
            pl.BlockSpec((BI, D), lambda i: (i, 0)),
            pl.BlockSpec((BI, N), lambda i: (i, 0)),
        ],
        out_shape=[
            jax.ShapeDtypeStruct((N, D), jnp.float32),
            jax.ShapeDtypeStruct((N, N), jnp.bfloat16),
        ],
        scratch_shapes=[pltpu.VMEM((N, D), jnp.float32)],
        compiler_params=pltpu.CompilerParams(
            dimension_semantics=("arbitrary",)),
    )(adj, h, W, b, a)


def _bf16_pass(adj_bf, h, W, b, a, n_out):
    F = W.shape[1]
    return pl.pallas_call(
        functools.partial(_bf16_pass_kernel, n_out),
        grid=(NI,),
        in_specs=[
            pl.BlockSpec((BI, N), lambda i: (i, 0)),
            pl.BlockSpec((N, D), lambda i: (0, 0)),
            pl.BlockSpec((D, F), lambda i: (0, 0)),
            pl.BlockSpec((1, F), lambda i: (0, 0)),
            pl.BlockSpec((1, F), lambda i: (0, 0)),
        ],
        out_specs=[pl.BlockSpec((BI, D), lambda i: (i, 0))] * n_out,
        out_shape=[jax.ShapeDtypeStruct((N, D), jnp.float32)] * n_out,
        scratch_shapes=[pltpu.VMEM((N, F), jnp.bfloat16)],
        compiler_params=pltpu.CompilerParams(
            dimension_semantics=("arbitrary",)),
    )(adj_bf, h, W, b, a)


def kernel(x, edge_index, Ws, bs, alphas):
    adj = edge_index
    h, adj_bf = _first_pass(adj, x, Ws[0], bs[0][None, :],
                            jnp.full((1, D), alphas[0], dtype=jnp.float32))
    for i in (1, 2):
        (h,) = _bf16_pass(adj_bf, h, Ws[i], bs[i][None, :],
                          jnp.full((1, D), alphas[i], dtype=jnp.float32), 1)
    Wcat = jnp.concatenate([Ws[3], Ws[4], Ws[5], Ws[6]], axis=1)
    bcat = jnp.concatenate([bs[3], bs[4], bs[5], bs[6]])[None, :]
    acat = jnp.repeat(alphas[3:7], D)[None, :]
    outs = _bf16_pass(adj_bf, h, Wcat, bcat, acat, 4)
    return tuple(outs)


# speedup vs baseline: 2.6676x; 1.5392x over previous
import functools

import jax
import jax.numpy as jnp
from jax.experimental import pallas as pl
from jax.experimental.pallas import tpu as pltpu

N = 10000
D = 128
BI0 = 200
NI0 = N // BI0
BI8 = 1000
NI8 = N // BI8

_F8 = jnp.float8_e4m3fn
_ADJ_SCALE = 256.0
_SEQ_CAP = 384.0
_SEQ_EXT = D + 8
_CH = 2000


def _first_pass_kernel(adj_ref, h_ref, w_ref, b_ref, a_ref,
                       o_ref, adj_f8_ref, rowsum_ref, seq_ref):
    i = pl.program_id(0)

    @pl.when(i == 0)
    def _():
        for r in range(0, N, _CH):
            seq_ref[pl.ds(r, _CH), :D] = jnp.dot(
                h_ref[pl.ds(r, _CH), :], w_ref[...],
                preferred_element_type=jnp.float32)
        seq_ref[:, D:] = jnp.ones((N, _SEQ_EXT - D), dtype=jnp.float32)

    adj_blk = adj_ref[...]
    adj_f8_ref[...] = (adj_blk * _ADJ_SCALE).astype(_F8)
    zx = jnp.dot(adj_blk, seq_ref[...], preferred_element_type=jnp.float32)
    rowsum_ref[...] = zx[:, D:D + 1]
    z = zx[:, :D] + b_ref[...]
    z = jnp.where(z >= 0, z, a_ref[...] * z)
    o_ref[...] = jnp.maximum(z, 0.0)


def _f8_pass_kernel(n_out, adj_ref, rs_ref, h_ref, w_ref, b_ref, a_ref,
                    *refs):
    o_refs = refs[:n_out]
    seq_ref = refs[n_out]
    corr_ref = refs[n_out + 1]
    c1_ref = refs[n_out + 2]
    i = pl.program_id(0)

    F = corr_ref.shape[1]

    @pl.when(i == 0)
    def _():
        m = jnp.float32(1e-30)
        for r in range(0, N, _CH):
            sc = jnp.dot(h_ref[pl.ds(r, _CH), :], w_ref[...],
                         preferred_element_type=jnp.float32)
            m = jnp.maximum(m, jnp.max(jnp.abs(sc)))
        s = _SEQ_CAP / m
        esum = jnp.zeros((1, F), dtype=jnp.float32)
        for r in range(0, N, _CH):
            sc = jnp.dot(h_ref[pl.ds(r, _CH), :], w_ref[...],
                         preferred_element_type=jnp.float32) * s
            s8 = sc.astype(_F8)
            seq_ref[pl.ds(r, _CH), :] = s8
            esum += jnp.sum(s8.astype(jnp.float32) - sc,
                            axis=0, keepdims=True)
        corr_ref[...] = esum * (1.0 / (N * s))
        c1_ref[0, 0] = 1.0 / (_ADJ_SCALE * s)

    z = jnp.dot(adj_ref[...], seq_ref[...],
                preferred_element_type=jnp.float32)
    rs_blk = rs_ref[pl.ds(i * BI8, BI8), :]
    z = z * c1_ref[0, 0] - rs_blk * corr_ref[...] + b_ref[...]
    z = jnp.where(z >= 0, z, a_ref[...] * z)
    z = jnp.maximum(z, 0.0)
    for j in range(n_out):
        o_refs[j][...] = z[:, j * D:(j + 1) * D]


def _first_pass(adj, h, W, b, a):
    return pl.pallas_call(
        _first_pass_kernel,
        grid=(NI0,),
        in_specs=[
            pl.BlockSpec((BI0, N), lambda i: (i, 0)),
            pl.BlockSpec((N, D), lambda i: (0, 0)),
            pl.BlockSpec((D, D), lambda i: (0, 0)),
            pl.BlockSpec((1, D), lambda i: (0, 0)),
            pl.BlockSpec((1, D), lambda i: (0, 0)),
        ],
        out_specs=[
            pl.BlockSpec((BI0, D), lambda i: (i, 0)),
            pl.BlockSpec((BI0, N), lambda i: (i, 0)),
            pl.BlockSpec((BI0, 1), lambda i: (i, 0)),
        ],
        out_shape=[
            jax.ShapeDtypeStruct((N, D), jnp.float32),
            jax.ShapeDtypeStruct((N, N), _F8),
            jax.ShapeDtypeStruct((N, 1), jnp.float32),
        ],
        scratch_shapes=[pltpu.VMEM((N, _SEQ_EXT), jnp.float32)],
        compiler_params=pltpu.CompilerParams(
            dimension_semantics=("arbitrary",)),
    )(adj, h, W, b, a)


def _f8_pass(adj_f8, rowsums, h, W, b, a, n_out):
    F = W.shape[1]
    return pl.pallas_call(
        functools.partial(_f8_pass_kernel, n_out),
        grid=(NI8,),
        in_specs=[
            pl.BlockSpec((BI8, N), lambda i: (i, 0)),
            pl.BlockSpec((N, 1), lambda i: (0, 0)),
            pl.BlockSpec((N, D), lambda i: (0, 0)),
            pl.BlockSpec((D, F), lambda i: (0, 0)),
            pl.BlockSpec((1, F), lambda i: (0, 0)),
            pl.BlockSpec((1, F), lambda i: (0, 0)),
        ],
        out_specs=[pl.BlockSpec((BI8, D), lambda i: (i, 0))] * n_out,
        out_shape=[jax.ShapeDtypeStruct((N, D), jnp.float32)] * n_out,
        scratch_shapes=[pltpu.VMEM((N, F), _F8),
                        pltpu.VMEM((1, F), jnp.float32),
                        pltpu.SMEM((1, 1), jnp.float32)],
        compiler_params=pltpu.CompilerParams(
            dimension_semantics=("arbitrary",)),
    )(adj_f8, rowsums, h, W, b, a)


def kernel(x, edge_index, Ws, bs, alphas):
    adj = edge_index
    h, adj_f8, rowsums = _first_pass(
        adj, x, Ws[0], bs[0][None, :],
        jnp.full((1, D), alphas[0], dtype=jnp.float32))
    for i in (1, 2):
        (h,) = _f8_pass(adj_f8, rowsums, h, Ws[i], bs[i][None, :],
                        jnp.full((1, D), alphas[i], dtype=jnp.float32), 1)
    Wcat = jnp.concatenate([Ws[3], Ws[4], Ws[5], Ws[6]], axis=1)
    bcat = jnp.concatenate([bs[3], bs[4], bs[5], bs[6]])[None, :]
    acat = jnp.repeat(alphas[3:7], D)[None, :]
    outs = _f8_pass(adj_f8, rowsums, h, Wcat, bcat, acat, 4)
    return tuple(outs)


# speedup vs baseline: 2.7078x; 1.0151x over previous
import functools

import jax
import jax.numpy as jnp
from jax.experimental import pallas as pl
from jax.experimental.pallas import tpu as pltpu

N = 10000
D = 128
BI0 = 400
NI0 = N // BI0
BI8_NARROW = 1000
BI8_WIDE = 1000

_F8 = jnp.float8_e4m3fn
_ADJ_SCALE = 256.0
_SEQ_CAP = 384.0
_SEQ_EXT = D + 8
_CH = 2000


def _first_pass_kernel(adj_ref, h_ref, w_ref, b_ref, a_ref,
                       o_ref, adj_f8_ref, rowsum_ref, seq_ref):
    i = pl.program_id(0)

    @pl.when(i == 0)
    def _():
        for r in range(0, N, _CH):
            seq_ref[pl.ds(r, _CH), :D] = jnp.dot(
                h_ref[pl.ds(r, _CH), :], w_ref[...],
                preferred_element_type=jnp.float32)
        seq_ref[:, D:] = jnp.ones((N, _SEQ_EXT - D), dtype=jnp.float32)

    adj_blk = adj_ref[...]
    adj_f8_ref[...] = (adj_blk * _ADJ_SCALE).astype(_F8)
    zx = jnp.dot(adj_blk, seq_ref[...], preferred_element_type=jnp.float32)
    rowsum_ref[...] = zx[:, D:D + 1]
    z = zx[:, :D] + b_ref[...]
    z = jnp.where(z >= 0, z, a_ref[...] * z)
    o_ref[...] = jnp.maximum(z, 0.0)


def _f8_pass_kernel(n_out, bi, adj_ref, rs_ref, h_ref, w_ref, b_ref, a_ref,
                    *refs):
    o_refs = refs[:n_out]
    seq_ref = refs[n_out]
    corr_ref = refs[n_out + 1]
    c1_ref = refs[n_out + 2]
    i = pl.program_id(0)

    F = corr_ref.shape[1]

    @pl.when(i == 0)
    def _():
        m = jnp.float32(1e-30)
        for r in range(0, N, _CH):
            sc = jnp.dot(h_ref[pl.ds(r, _CH), :], w_ref[...],
                         preferred_element_type=jnp.float32)
            m = jnp.maximum(m, jnp.max(jnp.abs(sc)))
        s = _SEQ_CAP / m
        esum = jnp.zeros((1, F), dtype=jnp.float32)
        for r in range(0, N, _CH):
            sc = jnp.dot(h_ref[pl.ds(r, _CH), :], w_ref[...],
                         preferred_element_type=jnp.float32) * s
            s8 = sc.astype(_F8)
            seq_ref[pl.ds(r, _CH), :] = s8
            esum += jnp.sum(s8.astype(jnp.float32) - sc,
                            axis=0, keepdims=True)
        corr_ref[...] = esum * (1.0 / (N * s))
        c1_ref[0, 0] = 1.0 / (_ADJ_SCALE * s)

    z = jnp.dot(adj_ref[...], seq_ref[...],
                preferred_element_type=jnp.float32)
    rs_blk = rs_ref[pl.ds(i * bi, bi), :]
    z = z * c1_ref[0, 0] - rs_blk * corr_ref[...] + b_ref[...]
    z = jnp.where(z >= 0, z, a_ref[...] * z)
    z = jnp.maximum(z, 0.0)
    for j in range(n_out):
        o_refs[j][...] = z[:, j * D:(j + 1) * D]


def _first_pass(adj, h, W, b, a):
    return pl.pallas_call(
        _first_pass_kernel,
        grid=(NI0,),
        in_specs=[
            pl.BlockSpec((BI0, N), lambda i: (i, 0)),
            pl.BlockSpec((N, D), lambda i: (0, 0)),
            pl.BlockSpec((D, D), lambda i: (0, 0)),
            pl.BlockSpec((1, D), lambda i: (0, 0)),
            pl.BlockSpec((1, D), lambda i: (0, 0)),
        ],
        out_specs=[
            pl.BlockSpec((BI0, D), lambda i: (i, 0)),
            pl.BlockSpec((BI0, N), lambda i: (i, 0)),
            pl.BlockSpec((BI0, 1), lambda i: (i, 0)),
        ],
        out_shape=[
            jax.ShapeDtypeStruct((N, D), jnp.float32),
            jax.ShapeDtypeStruct((N, N), _F8),
            jax.ShapeDtypeStruct((N, 1), jnp.float32),
        ],
        scratch_shapes=[pltpu.VMEM((N, _SEQ_EXT), jnp.float32)],
        compiler_params=pltpu.CompilerParams(
            dimension_semantics=("arbitrary",)),
    )(adj, h, W, b, a)


def _f8_pass(adj_f8, rowsums, h, W, b, a, n_out):
    F = W.shape[1]
    bi = BI8_WIDE if n_out > 1 else BI8_NARROW
    return pl.pallas_call(
        functools.partial(_f8_pass_kernel, n_out, bi),
        grid=(N // bi,),
        in_specs=[
            pl.BlockSpec((bi, N), lambda i: (i, 0)),
            pl.BlockSpec((N, 1), lambda i: (0, 0)),
            pl.BlockSpec((N, D), lambda i: (0, 0)),
            pl.BlockSpec((D, F), lambda i: (0, 0)),
            pl.BlockSpec((1, F), lambda i: (0, 0)),
            pl.BlockSpec((1, F), lambda i: (0, 0)),
        ],
        out_specs=[pl.BlockSpec((bi, D), lambda i: (i, 0))] * n_out,
        out_shape=[jax.ShapeDtypeStruct((N, D), jnp.float32)] * n_out,
        scratch_shapes=[pltpu.VMEM((N, F), _F8),
                        pltpu.VMEM((1, F), jnp.float32),
                        pltpu.SMEM((1, 1), jnp.float32)],
        compiler_params=pltpu.CompilerParams(
            dimension_semantics=("arbitrary",)),
    )(adj_f8, rowsums, h, W, b, a)


def kernel(x, edge_index, Ws, bs, alphas):
    adj = edge_index
    h, adj_f8, rowsums = _first_pass(
        adj, x, Ws[0], bs[0][None, :],
        jnp.full((1, D), alphas[0], dtype=jnp.float32))
    for i in (1, 2):
        (h,) = _f8_pass(adj_f8, rowsums, h, Ws[i], bs[i][None, :],
                        jnp.full((1, D), alphas[i], dtype=jnp.float32), 1)
    Wcat = jnp.concatenate([Ws[3], Ws[4], Ws[5], Ws[6]], axis=1)
    bcat = jnp.concatenate([bs[3], bs[4], bs[5], bs[6]])[None, :]
    acat = jnp.repeat(alphas[3:7], D)[None, :]
    outs = _f8_pass(adj_f8, rowsums, h, Wcat, bcat, acat, 4)
    return tuple(outs)
